# baseline (speedup 1.0000x reference)
import jax
import jax.numpy as jnp
from jax.experimental import pallas as pl
from jax.experimental.pallas import tpu as pltpu


def _make_body(BI):
    def body(x_ref, adj_ref, w1_ref, b1_ref, w2_ref, b2_ref, o_ref,
             s1_ref, g_ref):
        p = pl.program_id(0)
        i = pl.program_id(1)

        @pl.when(jnp.logical_and(p == 0, i == 0))
        def _():
            s1_ref[...] = jnp.dot(x_ref[...], w1_ref[...],
                                  preferred_element_type=jnp.float32)

        @pl.when(p == 0)
        def _():
            t = jnp.dot(adj_ref[...], s1_ref[...],
                        preferred_element_type=jnp.float32)
            h = jnp.maximum(t + b1_ref[...], 0.0)
            g_ref[pl.ds(i * BI, BI), :] = jnp.dot(
                h, w2_ref[...], preferred_element_type=jnp.float32)

        @pl.when(p == 1)
        def _():
            o_ref[...] = jnp.dot(adj_ref[...], g_ref[...],
                                 preferred_element_type=jnp.float32) \
                + b2_ref[...]

    return body


def kernel(x, adj, W1, b1, W2, b2):
    N, F = x.shape
    H = W1.shape[1]
    C = W2.shape[1]

    BI = 200
    assert N % BI == 0
    NI = N // BI

    b1r = b1.reshape(1, H)
    b2r = b2.reshape(1, C)

    out = pl.pallas_call(
        _make_body(BI),
        grid=(2, NI),
        in_specs=[
            pl.BlockSpec((N, F), lambda p, i: (0, 0)),
            pl.BlockSpec((BI, N), lambda p, i: (i, 0)),
            pl.BlockSpec((F, H), lambda p, i: (0, 0)),
            pl.BlockSpec((1, H), lambda p, i: (0, 0)),
            pl.BlockSpec((H, C), lambda p, i: (0, 0)),
            pl.BlockSpec((1, C), lambda p, i: (0, 0)),
        ],
        out_specs=pl.BlockSpec((BI, C), lambda p, i: (i * p, 0)),
        out_shape=jax.ShapeDtypeStruct((N, C), jnp.float32),
        scratch_shapes=[
            pltpu.VMEM((N, H), jnp.float32),
            pltpu.VMEM((N, C), jnp.float32),
        ],
        compiler_params=pltpu.CompilerParams(
            dimension_semantics=("arbitrary", "arbitrary"),
        ),
    )(x, adj, W1, b1r, W2, b2r)

    return out


# speedup vs baseline: 1.0159x; 1.0159x over previous
import jax
import jax.numpy as jnp
from jax.experimental import pallas as pl
from jax.experimental.pallas import tpu as pltpu


def _make_body(BI):
    def body(x_ref, adj_ref, w1_ref, b1_ref, w2_ref, b2_ref, o_ref,
             s1_ref, g_ref):
        p = pl.program_id(0)
        i = pl.program_id(1)

        @pl.when(jnp.logical_and(p == 0, i == 0))
        def _():
            s1_ref[...] = jnp.dot(x_ref[...], w1_ref[...],
                                  preferred_element_type=jnp.float32
                                  ).astype(jnp.bfloat16)

        @pl.when(p == 0)
        def _():
            a = adj_ref[...].astype(jnp.bfloat16)
            t = jnp.dot(a, s1_ref[...],
                        preferred_element_type=jnp.float32)
            h = jnp.maximum(t + b1_ref[...], 0.0)
            g_ref[pl.ds(i * BI, BI), :] = jnp.dot(
                h.astype(jnp.bfloat16),
                w2_ref[...].astype(jnp.bfloat16),
                preferred_element_type=jnp.float32).astype(jnp.bfloat16)

        @pl.when(p == 1)
        def _():
            a = adj_ref[...].astype(jnp.bfloat16)
            o_ref[...] = jnp.dot(a, g_ref[...],
                                 preferred_element_type=jnp.float32) \
                + b2_ref[...]

    return body


def kernel(x, adj, W1, b1, W2, b2):
    N, F = x.shape
    H = W1.shape[1]
    C = W2.shape[1]

    BI = 400
    assert N % BI == 0
    NI = N // BI

    b1r = b1.reshape(1, H)
    b2r = b2.reshape(1, C)

    out = pl.pallas_call(
        _make_body(BI),
        grid=(2, NI),
        in_specs=[
            pl.BlockSpec((N, F), lambda p, i: (0, 0)),
            pl.BlockSpec((BI, N), lambda p, i: (i, 0)),
            pl.BlockSpec((F, H), lambda p, i: (0, 0)),
            pl.BlockSpec((1, H), lambda p, i: (0, 0)),
            pl.BlockSpec((H, C), lambda p, i: (0, 0)),
            pl.BlockSpec((1, C), lambda p, i: (0, 0)),
        ],
        out_specs=pl.BlockSpec((BI, C), lambda p, i: (i * p, 0)),
        out_shape=jax.ShapeDtypeStruct((N, C), jnp.float32),
        scratch_shapes=[
            pltpu.VMEM((N, H), jnp.bfloat16),
            pltpu.VMEM((N, C), jnp.bfloat16),
        ],
        compiler_params=pltpu.CompilerParams(
            dimension_semantics=("arbitrary", "arbitrary"),
        ),
    )(x, adj, W1, b1r, W2, b2r)

    return out


# speedup vs baseline: 1.0241x; 1.0081x over previous
import jax
import jax.numpy as jnp
from jax.experimental import pallas as pl
from jax.experimental.pallas import tpu as pltpu


def _make_body(BI):
    def body(x_ref, adj_ref, w1_ref, b1_ref, w2_ref, b2_ref, o_ref,
             s1_ref, g_ref):
        p = pl.program_id(0)
        i = pl.program_id(1)

        @pl.when(jnp.logical_and(p == 0, i == 0))
        def _():
            s1_ref[...] = jnp.dot(x_ref[...], w1_ref[...],
                                  preferred_element_type=jnp.float32)

        @pl.when(p == 0)
        def _():
            t = jnp.dot(adj_ref[...], s1_ref[...],
                        preferred_element_type=jnp.float32)
            h = jnp.maximum(t + b1_ref[...], 0.0)
            g_ref[pl.ds(i * BI, BI), :] = jnp.dot(
                h, w2_ref[...], preferred_element_type=jnp.float32)

        @pl.when(p == 1)
        def _():
            o_ref[...] = jnp.dot(adj_ref[...], g_ref[...],
                                 preferred_element_type=jnp.float32) \
                + b2_ref[...]

    return body


def kernel(x, adj, W1, b1, W2, b2):
    N, F = x.shape
    H = W1.shape[1]
    C = W2.shape[1]

    BI = 400
    assert N % BI == 0
    NI = N // BI

    b1r = b1.reshape(1, H)
    b2r = b2.reshape(1, C)

    out = pl.pallas_call(
        _make_body(BI),
        grid=(2, NI),
        in_specs=[
            pl.BlockSpec((N, F), lambda p, i: (0, 0)),
            pl.BlockSpec((BI, N), lambda p, i: (i, 0)),
            pl.BlockSpec((F, H), lambda p, i: (0, 0)),
            pl.BlockSpec((1, H), lambda p, i: (0, 0)),
            pl.BlockSpec((H, C), lambda p, i: (0, 0)),
            pl.BlockSpec((1, C), lambda p, i: (0, 0)),
        ],
        out_specs=pl.BlockSpec((BI, C), lambda p, i: (i * p, 0)),
        out_shape=jax.ShapeDtypeStruct((N, C), jnp.float32),
        scratch_shapes=[
            pltpu.VMEM((N, H), jnp.float32),
            pltpu.VMEM((N, C), jnp.float32),
        ],
        compiler_params=pltpu.CompilerParams(
            dimension_semantics=("arbitrary", "arbitrary"),
        ),
    )(x, adj, W1, b1r, W2, b2r)

    return out
